# baseline (speedup 1.0000x reference)
import functools

import jax
import jax.numpy as jnp
import numpy as np
from jax import lax
from jax.experimental import pallas as pl
from jax.experimental.pallas import tpu as pltpu
from jax.experimental.pallas import tpu_sc as plsc

B, L, V, D = 4096, 200, 1000000, 64
LN = 16
NC, NS = 2, 16
NW = NC * NS
BPW = B // NW
C0, C1 = 104, 96
NACC = D // LN


_GATHER_DNUMS = lax.GatherDimensionNumbers(
    offset_dims=(), collapsed_slice_dims=(0,), start_index_map=(0,))


def _bcast_lane(vec, j):
    idx = jnp.full((LN, 1), j, dtype=jnp.int32)
    return lax.gather(vec, idx, dimension_numbers=_GATHER_DNUMS,
                      slice_sizes=(1,),
                      mode=lax.GatherScatterMode.PROMISE_IN_BOUNDS)


_mesh = plsc.VectorSubcoreMesh(core_axis_name="c", subcore_axis_name="s")


@functools.partial(
    pl.kernel,
    out_type=jax.ShapeDtypeStruct((B * D,), jnp.float32),
    mesh=_mesh,
    compiler_params=pltpu.CompilerParams(use_tc_tiling_on_sc=False),
    scratch_types=[
        pltpu.VMEM((BPW * L * 2,), jnp.float32),
        pltpu.VMEM((BPW * L + LN,), jnp.int32),
        pltpu.VMEM((2, L, D), jnp.float32),
        pltpu.VMEM((BPW * D,), jnp.float32),
        pltpu.SemaphoreType.DMA,
        pltpu.SemaphoreType.DMA,
    ],
)
def _embed_bag(x_hbm, emb_hbm, out_hbm,
               xv, idx_v, rows_v, out_v, sem0, sem1):
    wid = lax.axis_index("s") * NC + lax.axis_index("c")
    sems = (sem0, sem1)

    pltpu.sync_copy(
        x_hbm.at[pl.ds(pl.multiple_of(wid * (BPW * L * 2), 8), BPW * L * 2)],
        xv)

    lanes = lax.iota(jnp.int32, LN)
    perm_even = lax.broadcast_in_dim((lanes & 7) * 2, (LN, 1), (0,))
    lo_half = lanes < 8

    def _dgather(vec, perm):
        return lax.gather(vec, perm, dimension_numbers=_GATHER_DNUMS,
                          slice_sizes=(1,),
                          mode=lax.GatherScatterMode.PROMISE_IN_BOUNDS)

    def conv_body(i, carry):
        off = pl.multiple_of(i * (2 * LN), 8)
        v0 = xv[pl.ds(off, LN)]
        v1 = xv[pl.ds(pl.multiple_of(off + LN, 8), LN)]
        ivals = jnp.where(lo_half, _dgather(v0, perm_even),
                          _dgather(v1, perm_even))
        idx_v[pl.ds(pl.multiple_of(i * LN, 8), LN)] = ivals.astype(jnp.int32)
        return carry

    lax.fori_loop(0, BPW * L // LN, conv_body, 0)

    def start_gather(b, buf):
        off = pl.multiple_of(b * L, 8)
        pltpu.async_copy(
            emb_hbm.at[idx_v.at[pl.ds(off, C0)]],
            rows_v.at[buf, pl.ds(0, C0)], sems[buf])
        pltpu.async_copy(
            emb_hbm.at[idx_v.at[pl.ds(pl.multiple_of(off + C0, 8), C1)]],
            rows_v.at[buf, pl.ds(C0, C1)], sems[buf])

    def wait_gather(buf):
        pltpu.make_async_copy(emb_hbm.at[pl.ds(0, L)],
                              rows_v.at[buf], sems[buf]).wait()

    def compute(b, buf):
        accs = [jnp.zeros((LN,), jnp.float32) for _ in range(NACC)]
        for c in range(L // (LN // 2)):
            wv = xv[pl.ds(pl.multiple_of((b * L + c * (LN // 2)) * 2, 8), LN)]
            for j in range(LN // 2):
                wb = _bcast_lane(wv, 2 * j + 1)
                r = c * (LN // 2) + j
                for k in range(NACC):
                    accs[k] = accs[k] + wb * rows_v[buf, r, pl.ds(k * LN, LN)]

        obase = pl.multiple_of(b * D, 8)
        for k in range(NACC):
            out_v[pl.ds(pl.multiple_of(obase + k * LN, 8), LN)] = accs[k]

    start_gather(0, 0)
    start_gather(1, 1)

    def body2(i, carry):
        b0 = i * 2
        for buf in range(2):
            b = b0 + buf
            wait_gather(buf)
            compute(b, buf)
            nxt = b + 2

            @pl.when(nxt < BPW)
            def _():
                start_gather(nxt, buf)
        return carry

    lax.fori_loop(0, BPW // 2, body2, 0)

    pltpu.sync_copy(out_v,
                    out_hbm.at[pl.ds(pl.multiple_of(wid * (BPW * D), 8), BPW * D)])


def kernel(x, embedding):
    return _embed_bag(x.reshape(B * L * 2), embedding).reshape(B, D)


# speedup vs baseline: 1.2916x; 1.2916x over previous
import functools

import jax
import jax.numpy as jnp
from jax import lax
from jax.experimental import pallas as pl
from jax.experimental.pallas import tpu as pltpu
from jax.experimental.pallas import tpu_sc as plsc

B, L, V, D = 4096, 200, 1000000, 64
LN = 16
NC, NS = 2, 16
NW = NC * NS
TB = B // NW
XB = 2 * TB
NACC = D // LN


_GATHER_DNUMS = lax.GatherDimensionNumbers(
    offset_dims=(), collapsed_slice_dims=(0,), start_index_map=(0,))


def _bcast_lane(vec, j):
    idx = jnp.full((LN, 1), j, dtype=jnp.int32)
    return lax.gather(vec, idx, dimension_numbers=_GATHER_DNUMS,
                      slice_sizes=(1,),
                      mode=lax.GatherScatterMode.PROMISE_IN_BOUNDS)


_mesh = plsc.VectorSubcoreMesh(core_axis_name="c", subcore_axis_name="s")


@functools.partial(
    pl.kernel,
    out_type=jax.ShapeDtypeStruct((B * D,), jnp.float32),
    mesh=_mesh,
    compiler_params=pltpu.CompilerParams(use_tc_tiling_on_sc=False),
    scratch_types=[
        pltpu.VMEM((L * XB,), jnp.float32),
        pltpu.VMEM((L * TB,), jnp.int32),
        pltpu.VMEM((2, TB, D), jnp.float32),
        pltpu.VMEM((TB * D,), jnp.float32),
        pltpu.SemaphoreType.DMA,
        pltpu.SemaphoreType.DMA,
        pltpu.SemaphoreType.DMA,
    ],
)
def _embed_bag(x_hbm, emb_hbm, out_hbm,
               xv, idx_v, rows_v, acc_v, sem0, sem1, sem2):
    wid = lax.axis_index("s") * NC + lax.axis_index("c")
    sems = (sem0, sem1)

    def stage_body(l, carry):
        src = pl.multiple_of(l * (NW * XB) + wid * XB, 8)
        pltpu.async_copy(x_hbm.at[pl.ds(src, XB)],
                         xv.at[pl.ds(pl.multiple_of(l * XB, 8), XB)], sem2)
        return carry

    lax.fori_loop(0, L, stage_body, 0)
    pltpu.make_async_copy(x_hbm.at[pl.ds(0, L * XB)], xv, sem2).wait()

    def conv_body(i, carry):
        l, j = i // (TB // LN), i % (TB // LN)
        src = pl.multiple_of(l * XB + j * LN, 8)
        vals = xv[pl.ds(src, LN)]
        idx_v[pl.ds(pl.multiple_of(l * TB + j * LN, 8), LN)] = (
            vals.astype(jnp.int32))
        return carry

    lax.fori_loop(0, L * (TB // LN), conv_body, 0)

    zeros = jnp.zeros((LN,), jnp.float32)

    def zero_body(i, carry):
        acc_v[pl.ds(pl.multiple_of(i * LN, 8), LN)] = zeros
        return carry

    lax.fori_loop(0, TB * D // LN, zero_body, 0)

    def start_gather(l, buf):
        pltpu.async_copy(
            emb_hbm.at[idx_v.at[pl.ds(pl.multiple_of(l * TB, 8), TB)]],
            rows_v.at[buf], sems[buf])

    def wait_gather(buf):
        pltpu.make_async_copy(emb_hbm.at[pl.ds(0, TB)],
                              rows_v.at[buf], sems[buf]).wait()

    def accumulate(l, buf):
        for g in range(TB // LN):
            wv = xv[pl.ds(pl.multiple_of(l * XB + TB + g * LN, 8), LN)]
            for j in range(LN):
                bp = g * LN + j
                wb = _bcast_lane(wv, j)
                for k in range(NACC):
                    plsc.addupdate(
                        acc_v.at[pl.ds(pl.multiple_of(bp * D + k * LN, 8), LN)],
                        wb * rows_v[buf, bp, pl.ds(k * LN, LN)])

    start_gather(0, 0)
    start_gather(1, 1)

    def body2(i, carry):
        l0 = i * 2
        for buf in range(2):
            l = l0 + buf
            wait_gather(buf)
            accumulate(l, buf)
            nxt = l + 2

            @pl.when(nxt < L)
            def _():
                start_gather(nxt, buf)
        return carry

    lax.fori_loop(0, L // 2, body2, 0)

    pltpu.sync_copy(acc_v,
                    out_hbm.at[pl.ds(pl.multiple_of(wid * (TB * D), 8), TB * D)])


def kernel(x, embedding):
    xt = x.reshape(NW, TB, L, 2).transpose(2, 0, 3, 1).reshape(L * NW * XB)
    return _embed_bag(xt, embedding).reshape(B, D)


# speedup vs baseline: 2.0333x; 1.5742x over previous
import functools

import jax
import jax.numpy as jnp
from jax import lax
from jax.experimental import pallas as pl
from jax.experimental.pallas import tpu as pltpu
from jax.experimental.pallas import tpu_sc as plsc

B, L, V, D = 4096, 200, 1000000, 64
LN = 16
NC, NS = 2, 16
NW = NC * NS
BPW = B // NW
C0, C1 = 104, 96
NACC = D // LN


_GATHER_DNUMS = lax.GatherDimensionNumbers(
    offset_dims=(), collapsed_slice_dims=(0,), start_index_map=(0,))


def _bcast_lane(vec, j):
    idx = jnp.full((LN, 1), j, dtype=jnp.int32)
    return lax.gather(vec, idx, dimension_numbers=_GATHER_DNUMS,
                      slice_sizes=(1,),
                      mode=lax.GatherScatterMode.PROMISE_IN_BOUNDS)


_mesh = plsc.VectorSubcoreMesh(core_axis_name="c", subcore_axis_name="s")


@functools.partial(
    pl.kernel,
    out_type=jax.ShapeDtypeStruct((B * D,), jnp.float32),
    mesh=_mesh,
    compiler_params=pltpu.CompilerParams(use_tc_tiling_on_sc=False),
    scratch_types=[
        pltpu.VMEM((BPW * L,), jnp.int32),
        pltpu.VMEM((BPW * L,), jnp.float32),
        pltpu.VMEM((2, L, D), jnp.float32),
        pltpu.VMEM((BPW * D,), jnp.float32),
        pltpu.SemaphoreType.DMA,
        pltpu.SemaphoreType.DMA,
    ],
)
def _embed_bag(idx_hbm, w_hbm, emb_hbm, out_hbm,
               idx_v, w_v, rows_v, out_v, sem0, sem1):
    wid = lax.axis_index("s") * NC + lax.axis_index("c")
    sems = (sem0, sem1)

    pltpu.sync_copy(idx_hbm.at[pl.ds(pl.multiple_of(wid * (BPW * L), 8), BPW * L)],
                    idx_v)
    pltpu.sync_copy(w_hbm.at[pl.ds(pl.multiple_of(wid * (BPW * L), 8), BPW * L)],
                    w_v)

    def start_gather(b, buf):
        off = pl.multiple_of(b * L, 8)
        pltpu.async_copy(
            emb_hbm.at[idx_v.at[pl.ds(off, C0)]],
            rows_v.at[buf, pl.ds(0, C0)], sems[buf])
        pltpu.async_copy(
            emb_hbm.at[idx_v.at[pl.ds(pl.multiple_of(off + C0, 8), C1)]],
            rows_v.at[buf, pl.ds(C0, C1)], sems[buf])

    def wait_gather(buf):
        pltpu.make_async_copy(emb_hbm.at[pl.ds(0, L)],
                              rows_v.at[buf], sems[buf]).wait()

    def compute(b, buf):
        accs = [jnp.zeros((LN,), jnp.float32) for _ in range(NACC)]
        for c in range(L // LN):
            wv = w_v[pl.ds(pl.multiple_of(b * L + c * LN, 8), LN)]
            for j in range(LN):
                wb = _bcast_lane(wv, j)
                r = c * LN + j
                for k in range(NACC):
                    accs[k] = accs[k] + wb * rows_v[buf, r, pl.ds(k * LN, LN)]
        wv = w_v[pl.ds(pl.multiple_of(b * L + L - LN, 8), LN)]
        for j in range(LN - (L % LN), LN):
            wb = _bcast_lane(wv, j)
            r = L - LN + j
            for k in range(NACC):
                accs[k] = accs[k] + wb * rows_v[buf, r, pl.ds(k * LN, LN)]

        obase = pl.multiple_of(b * D, 8)
        for k in range(NACC):
            out_v[pl.ds(pl.multiple_of(obase + k * LN, 8), LN)] = accs[k]

    start_gather(0, 0)
    start_gather(1, 1)

    def body2(i, carry):
        b0 = i * 2
        for buf in range(2):
            b = b0 + buf
            wait_gather(buf)
            compute(b, buf)
            nxt = b + 2

            @pl.when(nxt < BPW)
            def _():
                start_gather(nxt, buf)
        return carry

    lax.fori_loop(0, BPW // 2, body2, 0)

    pltpu.sync_copy(out_v,
                    out_hbm.at[pl.ds(pl.multiple_of(wid * (BPW * D), 8), BPW * D)])


def kernel(x, embedding):
    xt = x.reshape(NW, BPW, L, 2).transpose(2, 0, 3, 1)
    idx = xt[:, :, 0, :].astype(jnp.int32).transpose(1, 2, 0).reshape(B * L)
    w = xt[:, :, 1, :].transpose(1, 2, 0).reshape(B * L)
    return _embed_bag(idx, w, embedding).reshape(B, D)


# speedup vs baseline: 2.0445x; 1.0055x over previous
import functools

import jax
import jax.numpy as jnp
from jax import lax
from jax.experimental import pallas as pl
from jax.experimental.pallas import tpu as pltpu
from jax.experimental.pallas import tpu_sc as plsc

B, L, V, D = 4096, 200, 1000000, 64
LN = 16
NC, NS = 2, 16
NW = NC * NS
BPW = B // NW
SL = 2 * BPW
C0, C1 = 104, 96
NACC = D // LN


_GATHER_DNUMS = lax.GatherDimensionNumbers(
    offset_dims=(), collapsed_slice_dims=(0,), start_index_map=(0,))


def _bcast_lane(vec, j):
    idx = jnp.full((LN, 1), j, dtype=jnp.int32)
    return lax.gather(vec, idx, dimension_numbers=_GATHER_DNUMS,
                      slice_sizes=(1,),
                      mode=lax.GatherScatterMode.PROMISE_IN_BOUNDS)


_mesh = plsc.VectorSubcoreMesh(core_axis_name="c", subcore_axis_name="s")


@functools.partial(
    pl.kernel,
    out_type=jax.ShapeDtypeStruct((B * D,), jnp.float32),
    mesh=_mesh,
    compiler_params=pltpu.CompilerParams(use_tc_tiling_on_sc=False),
    scratch_types=[
        pltpu.VMEM((BPW * SL,), jnp.int32),
        pltpu.VMEM((BPW * SL,), jnp.float32),
        pltpu.VMEM((2, L, D), jnp.float32),
        pltpu.VMEM((BPW * D,), jnp.float32),
        pltpu.SemaphoreType.DMA,
        pltpu.SemaphoreType.DMA,
    ],
)
def _embed_bag(idx_hbm, w_hbm, emb_hbm, out_hbm,
               idx_v, w_v, rows_v, out_v, sem0, sem1):
    wid = lax.axis_index("s") * NC + lax.axis_index("c")
    sems = (sem0, sem1)

    pltpu.sync_copy(idx_hbm.at[pl.ds(pl.multiple_of(wid * (BPW * SL), 8), BPW * SL)],
                    idx_v)
    pltpu.sync_copy(w_hbm.at[pl.ds(pl.multiple_of(wid * (BPW * SL), 8), BPW * SL)],
                    w_v)

    def start_gather(b, buf):
        off = pl.multiple_of(b * SL, 8)
        pltpu.async_copy(
            emb_hbm.at[idx_v.at[pl.ds(off, C0)]],
            rows_v.at[buf, pl.ds(0, C0)], sems[buf])
        pltpu.async_copy(
            emb_hbm.at[idx_v.at[pl.ds(pl.multiple_of(off + C0, 8), C1)]],
            rows_v.at[buf, pl.ds(C0, C1)], sems[buf])

    def wait_gather(buf):
        pltpu.make_async_copy(emb_hbm.at[pl.ds(0, L)],
                              rows_v.at[buf], sems[buf]).wait()

    def compute(b, buf):
        accs = [jnp.zeros((LN,), jnp.float32) for _ in range(NACC)]
        for c in range(L // LN):
            wv = w_v[pl.ds(pl.multiple_of(b * SL + c * LN, 8), LN)]
            for j in range(LN):
                wb = _bcast_lane(wv, j)
                r = c * LN + j
                for k in range(NACC):
                    accs[k] = accs[k] + wb * rows_v[buf, r, pl.ds(k * LN, LN)]
        wv = w_v[pl.ds(pl.multiple_of(b * SL + L - LN, 8), LN)]
        for j in range(LN - (L % LN), LN):
            wb = _bcast_lane(wv, j)
            r = L - LN + j
            for k in range(NACC):
                accs[k] = accs[k] + wb * rows_v[buf, r, pl.ds(k * LN, LN)]

        obase = pl.multiple_of(b * D, 8)
        for k in range(NACC):
            out_v[pl.ds(pl.multiple_of(obase + k * LN, 8), LN)] = accs[k]

    start_gather(0, 0)
    start_gather(1, 1)

    def body2(i, carry):
        b0 = i * 2
        for buf in range(2):
            b = b0 + buf
            wait_gather(buf)
            compute(b, buf)
            nxt = b + 2

            @pl.when(nxt < BPW)
            def _():
                start_gather(nxt, buf)
        return carry

    lax.fori_loop(0, BPW // 2, body2, 0)

    pltpu.sync_copy(out_v,
                    out_hbm.at[pl.ds(pl.multiple_of(wid * (BPW * D), 8), BPW * D)])


def kernel(x, embedding):
    idx = x[:, :, 0].astype(jnp.int32)
    w = x[:, :, 1]
    pad = ((0, 0), (0, SL - L))
    idx2 = jnp.pad(idx, pad).reshape(-1)
    w2 = jnp.pad(w, pad).reshape(-1)
    out = _embed_bag(idx2, w2, embedding)
    return out.reshape(B, D)
